# baseline (speedup 1.0000x reference)
import jax
import jax.numpy as jnp
from jax.experimental import pallas as pl
from jax.experimental.pallas import tpu as pltpu

_N = 20000
_THR = 0.5
_CW = 5120
_CHUNKS = [(o, min(_CW, _N - o)) for o in range(0, _N, _CW)]
_NC = len(_CHUNKS)


def _body(bt_hbm, s_hbm, o_hbm, bt_v, s_v, o_v, sin, sout):
    for i, (off, w) in enumerate(_CHUNKS):
        pltpu.make_async_copy(bt_hbm.at[:, pl.ds(off, w)],
                              bt_v.at[:, pl.ds(off, w)], sin.at[i]).start()
        pltpu.make_async_copy(s_hbm.at[:, pl.ds(off, w)],
                              s_v.at[:, pl.ds(off, w)], sin.at[i]).start()
    for i, (off, w) in enumerate(_CHUNKS):
        pltpu.make_async_copy(bt_hbm.at[:, pl.ds(off, w)],
                              bt_v.at[:, pl.ds(off, w)], sin.at[i]).wait()
        pltpu.make_async_copy(s_hbm.at[:, pl.ds(off, w)],
                              s_v.at[:, pl.ds(off, w)], sin.at[i]).wait()
        s = s_v[:, pl.ds(off, w)]
        bt = bt_v[:, pl.ds(off, w)]
        mf = jnp.where(s >= _THR, 1.0, 0.0).astype(jnp.float32)
        o_v[:, pl.ds(off, w)] = jnp.concatenate([bt * mf, s * mf], axis=0)
        pltpu.make_async_copy(o_v.at[:, pl.ds(off, w)],
                              o_hbm.at[:, pl.ds(off, w)], sout.at[i]).start()
    for i, (off, w) in enumerate(_CHUNKS):
        pltpu.make_async_copy(o_v.at[:, pl.ds(off, w)],
                              o_hbm.at[:, pl.ds(off, w)], sout.at[i]).wait()


_call = pl.pallas_call(
    _body,
    in_specs=[
        pl.BlockSpec(memory_space=pltpu.MemorySpace.HBM),
        pl.BlockSpec(memory_space=pltpu.MemorySpace.HBM),
    ],
    out_specs=pl.BlockSpec(memory_space=pltpu.MemorySpace.HBM),
    out_shape=jax.ShapeDtypeStruct((5, _N), jnp.float32),
    scratch_shapes=[
        pltpu.VMEM((4, _N), jnp.float32),
        pltpu.VMEM((1, _N), jnp.float32),
        pltpu.VMEM((5, _N), jnp.float32),
        pltpu.SemaphoreType.DMA((_NC,)),
        pltpu.SemaphoreType.DMA((_NC,)),
    ],
)


def kernel(keypoints, boxes, scores, labels):
    del keypoints, labels
    out5 = _call(boxes[0].T, scores)
    return out5.T


# speedup vs baseline: 1.0203x; 1.0203x over previous
import jax
import jax.numpy as jnp
from jax.experimental import pallas as pl
from jax.experimental.pallas import tpu as pltpu

_N = 20000
_THR = 0.5
_CW = 5120
_CHUNKS = [(o, min(_CW, _N - o)) for o in range(0, _N, _CW)]
_NC = len(_CHUNKS)


def _body(bt_hbm, s_hbm, o_hbm, bt_v, s_v, o_v, sin, sout):
    for i, (off, w) in enumerate(_CHUNKS):
        pltpu.make_async_copy(bt_hbm.at[:, pl.ds(off, w)],
                              bt_v.at[:, pl.ds(off, w)], sin.at[i]).start()
        pltpu.make_async_copy(s_hbm.at[:, pl.ds(off, w)],
                              s_v.at[:, pl.ds(off, w)], sin.at[i]).start()
    for i, (off, w) in enumerate(_CHUNKS):
        pltpu.make_async_copy(bt_hbm.at[:, pl.ds(off, w)],
                              bt_v.at[:, pl.ds(off, w)], sin.at[i]).wait()
        pltpu.make_async_copy(s_hbm.at[:, pl.ds(off, w)],
                              s_v.at[:, pl.ds(off, w)], sin.at[i]).wait()
        s = s_v[:, pl.ds(off, w)]
        bt = bt_v[:, pl.ds(off, w)]
        mf = jnp.where(s >= _THR, 1.0, 0.0).astype(jnp.float32)
        o_v[0:4, pl.ds(off, w)] = bt * mf
        o_v[4:5, pl.ds(off, w)] = s * mf
        pltpu.make_async_copy(o_v.at[:, pl.ds(off, w)],
                              o_hbm.at[:, pl.ds(off, w)], sout.at[i]).start()
    for i, (off, w) in enumerate(_CHUNKS):
        pltpu.make_async_copy(o_v.at[:, pl.ds(off, w)],
                              o_hbm.at[:, pl.ds(off, w)], sout.at[i]).wait()


_call = pl.pallas_call(
    _body,
    in_specs=[
        pl.BlockSpec(memory_space=pltpu.MemorySpace.HBM),
        pl.BlockSpec(memory_space=pltpu.MemorySpace.HBM),
    ],
    out_specs=pl.BlockSpec(memory_space=pltpu.MemorySpace.HBM),
    out_shape=jax.ShapeDtypeStruct((5, _N), jnp.float32),
    scratch_shapes=[
        pltpu.VMEM((4, _N), jnp.float32),
        pltpu.VMEM((1, _N), jnp.float32),
        pltpu.VMEM((5, _N), jnp.float32),
        pltpu.SemaphoreType.DMA((_NC,)),
        pltpu.SemaphoreType.DMA((_NC,)),
    ],
)


def kernel(keypoints, boxes, scores, labels):
    del keypoints, labels
    out5 = _call(boxes[0].T, scores)
    return out5.T
